# baseline (speedup 1.0000x reference)
import jax
import jax.numpy as jnp
from jax import lax
from jax.experimental import pallas as pl
from jax.experimental.pallas import tpu as pltpu
from jax.experimental.pallas import tpu_sc as plsc

N_NODES = 100000
DIM_EMB = 128
CHUNK = 400
NUM_CHUNKS = N_NODES // CHUNK
NUM_WORKERS = 32
CHUNKS_PER_WORKER = -(-NUM_CHUNKS // NUM_WORKERS)


def _emb_body(Z_hbm, table_hbm, out_hbm, idx_v, rows_v, sem):
    cid = lax.axis_index("c")
    sid = lax.axis_index("s")
    wid = sid * 2 + cid

    def step(i, carry):
        chunk = wid * CHUNKS_PER_WORKER + i

        @pl.when(chunk < NUM_CHUNKS)
        def _():
            base = chunk * CHUNK
            pltpu.sync_copy(Z_hbm.at[pl.ds(base, CHUNK)], idx_v)
            pltpu.async_copy(table_hbm.at[idx_v], rows_v, sem).wait()
            pltpu.sync_copy(rows_v, out_hbm.at[pl.ds(base, CHUNK)])

        return carry

    lax.fori_loop(0, CHUNKS_PER_WORKER, step, 0)


def kernel(Z, table):
    run = pl.kernel(
        _emb_body,
        out_type=jax.ShapeDtypeStruct((N_NODES, DIM_EMB), jnp.float32),
        mesh=plsc.VectorSubcoreMesh(core_axis_name="c", subcore_axis_name="s"),
        scratch_types=[
            pltpu.VMEM((CHUNK,), jnp.int32),
            pltpu.VMEM((CHUNK, DIM_EMB), jnp.float32),
            pltpu.SemaphoreType.DMA,
        ],
    )
    return run(Z, table)


# speedup vs baseline: 1.0063x; 1.0063x over previous
import jax
import jax.numpy as jnp
from jax import lax
from jax.experimental import pallas as pl
from jax.experimental.pallas import tpu as pltpu
from jax.experimental.pallas import tpu_sc as plsc

N_NODES = 100000
DIM_EMB = 128
CHUNK = 200
NUM_CHUNKS = N_NODES // CHUNK
NUM_WORKERS = 32
STEPS = -(-NUM_CHUNKS // NUM_WORKERS)
NBUF = 4


def _emb_body(Z_hbm, table_hbm, out_hbm, idx_v, rows_v,
              sem_idx, sem_g0, sem_g1, sem_g2, sem_g3,
              sem_w0, sem_w1, sem_w2, sem_w3):
    wid = lax.axis_index("s") * 2 + lax.axis_index("c")
    sem_g = (sem_g0, sem_g1, sem_g2, sem_g3)
    sem_w = (sem_w0, sem_w1, sem_w2, sem_w3)

    def chunk_of(i):
        return i * NUM_WORKERS + wid

    def guarded(i, fn):
        @pl.when(chunk_of(i) < NUM_CHUNKS)
        def _():
            fn()

    def start_idx(i):
        pltpu.make_async_copy(
            Z_hbm.at[pl.ds(chunk_of(i) * CHUNK, CHUNK)], idx_v.at[pl.ds(i * CHUNK, CHUNK)], sem_idx
        ).start()

    def wait_idx(i):
        pltpu.make_async_copy(
            Z_hbm.at[pl.ds(0, CHUNK)], idx_v.at[pl.ds(i * CHUNK, CHUNK)], sem_idx
        ).wait()

    def start_gather(i):
        b = i % NBUF
        pltpu.make_async_copy(
            table_hbm.at[idx_v.at[pl.ds(i * CHUNK, CHUNK)]], rows_v.at[b], sem_g[b]
        ).start()

    def wait_gather(i):
        b = i % NBUF
        pltpu.make_async_copy(
            table_hbm.at[idx_v.at[pl.ds(i * CHUNK, CHUNK)]], rows_v.at[b], sem_g[b]
        ).wait()

    def start_write(i):
        b = i % NBUF
        pltpu.make_async_copy(
            rows_v.at[b], out_hbm.at[pl.ds(chunk_of(i) * CHUNK, CHUNK)], sem_w[b]
        ).start()

    def wait_write(i):
        b = i % NBUF
        pltpu.make_async_copy(
            rows_v.at[b], out_hbm.at[pl.ds(0, CHUNK)], sem_w[b]
        ).wait()

    for i in range(STEPS):
        guarded(i, lambda i=i: start_idx(i))
    for i in range(STEPS):
        guarded(i, lambda i=i: wait_idx(i))

    for i in range(NBUF):
        guarded(i, lambda i=i: start_gather(i))
    for i in range(STEPS):
        guarded(i, lambda i=i: wait_gather(i))
        guarded(i, lambda i=i: start_write(i))
        if i + NBUF < STEPS:
            guarded(i, lambda i=i: wait_write(i))
            guarded(i + NBUF, lambda i=i: start_gather(i + NBUF))
    for i in range(STEPS - NBUF, STEPS):
        guarded(i, lambda i=i: wait_write(i))


def kernel(Z, table):
    run = pl.kernel(
        _emb_body,
        out_type=jax.ShapeDtypeStruct((N_NODES, DIM_EMB), jnp.float32),
        mesh=plsc.VectorSubcoreMesh(core_axis_name="c", subcore_axis_name="s"),
        scratch_types=[
            pltpu.VMEM((STEPS * CHUNK,), jnp.int32),
            pltpu.VMEM((NBUF, CHUNK, DIM_EMB), jnp.float32),
        ] + [pltpu.SemaphoreType.DMA] * 9,
    )
    return run(Z, table)


# speedup vs baseline: 11.2087x; 11.1384x over previous
import jax
import jax.numpy as jnp
from jax import lax
from jax.experimental import pallas as pl
from jax.experimental.pallas import tpu as pltpu
from jax.experimental.pallas import tpu_sc as plsc

N_NODES = 100000
DIM_EMB = 128
CHUNK = 200
NUM_CHUNKS = N_NODES // CHUNK
NUM_WORKERS = 32
STEPS = -(-NUM_CHUNKS // NUM_WORKERS)
NBUF = 4


def _emb_body(Z_hbm, table_hbm, out_hbm, idx_v, rows_v, table_sp,
              sem_idx, sem_g0, sem_g1, sem_g2, sem_g3,
              sem_w0, sem_w1, sem_w2, sem_w3):
    sid = lax.axis_index("s")
    wid = sid * 2 + lax.axis_index("c")
    sem_g = (sem_g0, sem_g1, sem_g2, sem_g3)
    sem_w = (sem_w0, sem_w1, sem_w2, sem_w3)

    @pl.when(sid == 0)
    def _():
        pltpu.sync_copy(table_hbm, table_sp)
    plsc.subcore_barrier()

    def chunk_of(i):
        return i * NUM_WORKERS + wid

    def guarded(i, fn):
        @pl.when(chunk_of(i) < NUM_CHUNKS)
        def _():
            fn()

    def start_idx(i):
        pltpu.make_async_copy(
            Z_hbm.at[pl.ds(chunk_of(i) * CHUNK, CHUNK)], idx_v.at[pl.ds(i * CHUNK, CHUNK)], sem_idx
        ).start()

    def wait_idx(i):
        pltpu.make_async_copy(
            Z_hbm.at[pl.ds(0, CHUNK)], idx_v.at[pl.ds(i * CHUNK, CHUNK)], sem_idx
        ).wait()

    def start_gather(i):
        b = i % NBUF
        pltpu.make_async_copy(
            table_sp.at[idx_v.at[pl.ds(i * CHUNK, CHUNK)]], rows_v.at[b], sem_g[b]
        ).start()

    def wait_gather(i):
        b = i % NBUF
        pltpu.make_async_copy(
            table_sp.at[idx_v.at[pl.ds(i * CHUNK, CHUNK)]], rows_v.at[b], sem_g[b]
        ).wait()

    def start_write(i):
        b = i % NBUF
        pltpu.make_async_copy(
            rows_v.at[b], out_hbm.at[pl.ds(chunk_of(i) * CHUNK, CHUNK)], sem_w[b]
        ).start()

    def wait_write(i):
        b = i % NBUF
        pltpu.make_async_copy(
            rows_v.at[b], out_hbm.at[pl.ds(0, CHUNK)], sem_w[b]
        ).wait()

    for i in range(STEPS):
        guarded(i, lambda i=i: start_idx(i))
    for i in range(STEPS):
        guarded(i, lambda i=i: wait_idx(i))

    for i in range(NBUF):
        guarded(i, lambda i=i: start_gather(i))
    for i in range(STEPS):
        guarded(i, lambda i=i: wait_gather(i))
        guarded(i, lambda i=i: start_write(i))
        if i + NBUF < STEPS:
            guarded(i, lambda i=i: wait_write(i))
            guarded(i + NBUF, lambda i=i: start_gather(i + NBUF))
    for i in range(STEPS - NBUF, STEPS):
        guarded(i, lambda i=i: wait_write(i))


def kernel(Z, table):
    run = pl.kernel(
        _emb_body,
        out_type=jax.ShapeDtypeStruct((N_NODES, DIM_EMB), jnp.float32),
        mesh=plsc.VectorSubcoreMesh(core_axis_name="c", subcore_axis_name="s"),
        scratch_types=[
            pltpu.VMEM((STEPS * CHUNK,), jnp.int32),
            pltpu.VMEM((NBUF, CHUNK, DIM_EMB), jnp.float32),
            pltpu.VMEM_SHARED((20, DIM_EMB), jnp.float32),
        ] + [pltpu.SemaphoreType.DMA] * 9,
    )
    return run(Z, table)
